# baseline (speedup 1.0000x reference)
import functools

import jax
import jax.numpy as jnp
from jax import lax
from jax.experimental import pallas as pl
from jax.experimental.pallas import tpu as pltpu
from jax.experimental.pallas import tpu_sc as plsc

NC = 2
NS = 16
NW = NC * NS
CH = 128


def _zero_rows(zbuf, nrows, width):
  z16 = jnp.zeros((16,), jnp.float32)

  def body(i, c):
    for j in range(width // 16):
      zbuf[i, pl.ds(j * 16, 16)] = z16
    return c

  lax.fori_loop(0, nrows, body, 0)


def _make_deg_kernel(npad, rpt, nchunks):
  mesh = plsc.VectorSubcoreMesh(core_axis_name="c", subcore_axis_name="s")
  zr = rpt // 4

  def body(dst_hbm, out_hbm, didx_all, ones_v, zbuf, deg_sh):
    cid = lax.axis_index("c")
    sid = lax.axis_index("s")
    wid = cid * NS + sid

    pltpu.sync_copy(dst_hbm.at[wid], didx_all)

    o16 = jnp.ones((16,), jnp.float32)

    def fill(i, c):
      ones_v[i, :] = o16
      return c

    lax.fori_loop(0, CH, fill, 0)
    _zero_rows(zbuf, zr, 16)

    def zcopy(i, c):
      pltpu.sync_copy(zbuf, deg_sh.at[pl.ds(sid * rpt + i * zr, zr)])
      return c

    lax.fori_loop(0, rpt // zr, zcopy, 0)
    plsc.subcore_barrier()

    def step(i, c):
      pltpu.sync_copy(ones_v, deg_sh.at[didx_all.at[i]], add=True)
      return c

    lax.fori_loop(0, nchunks, step, 0)
    plsc.subcore_barrier()
    pltpu.sync_copy(deg_sh.at[pl.ds(sid * rpt, rpt)],
                    out_hbm.at[cid, pl.ds(sid * rpt, rpt), :])

  return pl.kernel(
      body,
      out_type=jax.ShapeDtypeStruct((NC, npad, 16), jnp.float32),
      mesh=mesh,
      compiler_params=pltpu.CompilerParams(use_tc_tiling_on_sc=False),
      scratch_types=[
          pltpu.VMEM((nchunks, CH), jnp.int32),
          pltpu.VMEM((CH, 16), jnp.float32),
          pltpu.VMEM((zr, 16), jnp.float32),
          pltpu.VMEM_SHARED((npad, 16), jnp.float32),
      ],
  )


def _make_agg_kernel(npad, rpt, nchunks, nh, ch):
  mesh = plsc.VectorSubcoreMesh(core_axis_name="c", subcore_axis_name="s")
  hw = 64

  nb = 6
  gd = nb // 2
  assert nchunks % nb == 0 and nchunks > nb

  def body(*refs):
    g_halves = refs[:nh]
    src_hbm, dst_hbm, out_hbm = refs[nh:nh + 3]
    sidx_all, didx_all = refs[nh + 3:nh + 5]
    rows = refs[nh + 5:nh + 5 + nb]
    table, accum = refs[nh + 5 + nb:nh + 7 + nb]
    gsem = refs[nh + 7 + nb:nh + 7 + 2 * nb]
    ssem = refs[nh + 7 + 2 * nb:nh + 7 + 3 * nb]
    rows0 = rows[0]
    cid = lax.axis_index("c")
    sid = lax.axis_index("s")
    wid = cid * NS + sid

    pltpu.sync_copy(src_hbm.at[wid], sidx_all)
    pltpu.sync_copy(dst_hbm.at[wid], didx_all)

    for h in range(nh):
      if h:
        plsc.subcore_barrier()
      _zero_rows(rows0, ch, hw)
      pltpu.sync_copy(g_halves[h].at[pl.ds(sid * rpt, rpt)],
                      table.at[pl.ds(sid * rpt, rpt)])

      def zcopy(i, c):
        pltpu.sync_copy(rows0, accum.at[pl.ds(sid * rpt + i * ch, ch)])
        return c

      lax.fori_loop(0, rpt // ch, zcopy, 0)
      rem = rpt % ch
      if rem:
        pltpu.sync_copy(rows0.at[pl.ds(0, rem)],
                        accum.at[pl.ds(sid * rpt + (rpt // ch) * ch, rem)])
      plsc.subcore_barrier()

      def fire_gather(i, b):
        pltpu.async_copy(table.at[sidx_all.at[i]], rows[b], gsem[b])

      def wait_gather(i, b):
        pltpu.make_async_copy(table.at[sidx_all.at[i]], rows[b],
                              gsem[b]).wait()

      def fire_scatter(i, b):
        pltpu.async_copy(rows[b], accum.at[didx_all.at[i]], ssem[b],
                         add=True)

      def wait_scatter(i, b):
        pltpu.make_async_copy(rows[b], accum.at[didx_all.at[i]],
                              ssem[b]).wait()

      for i in range(gd):
        fire_gather(i, i)
      for i in range(nb - gd):
        wait_gather(i, i)
        fire_scatter(i, i)
        fire_gather(i + gd, (i + gd) % nb)

      def group(q, c):
        i0 = (nb - gd) + nb * q
        for j in range(nb):
          i = i0 + j
          b = (nb - gd + j) % nb
          wait_gather(i, b)
          fire_scatter(i, b)
          b2 = (b + gd) % nb
          wait_scatter(i + gd - nb, b2)
          fire_gather(i + gd, b2)
        return c

      lax.fori_loop(0, (nchunks - nb) // nb, group, 0)
      for i in range(nchunks - gd, nchunks):
        wait_gather(i, i % nb)
        fire_scatter(i, i % nb)
      for i in range(nchunks - nb, nchunks):
        wait_scatter(i, i % nb)
      plsc.subcore_barrier()
      pltpu.sync_copy(accum.at[pl.ds(sid * rpt, rpt)],
                      out_hbm.at[cid, h, pl.ds(sid * rpt, rpt), :])

  return pl.kernel(
      body,
      out_type=jax.ShapeDtypeStruct((NC, nh, npad, hw), jnp.float32),
      mesh=mesh,
      compiler_params=pltpu.CompilerParams(use_tc_tiling_on_sc=False),
      scratch_types=(
          [pltpu.VMEM((nchunks, ch), jnp.int32)] * 2 +
          [pltpu.VMEM((ch, hw), jnp.float32)] * nb +
          [pltpu.VMEM_SHARED((npad, hw), jnp.float32)] * 2 +
          [pltpu.SemaphoreType.DMA] * (2 * nb)),
  )


def _prep1(x_p, d0, d1, w1, npad, br, k, hid):
  def body(x_ref, d0_ref, d1_ref, w_ref, glo_ref, ghi_ref, dinv_ref):
    deg = d0_ref[...] + d1_ref[...] + 1.0
    dinv = lax.rsqrt(deg)
    dinv_ref[...] = dinv
    xs = x_ref[...] * dinv[:, 0:1]
    g = lax.dot_general(xs, w_ref[...], (((1,), (1,)), ((), ())),
                        preferred_element_type=jnp.float32)
    glo_ref[...] = g[:, :hid // 2]
    ghi_ref[...] = g[:, hid // 2:]

  return pl.pallas_call(
      body,
      grid=(npad // br,),
      in_specs=[
          pl.BlockSpec((br, k), lambda i: (i, 0)),
          pl.BlockSpec((br, 16), lambda i: (i, 0)),
          pl.BlockSpec((br, 16), lambda i: (i, 0)),
          pl.BlockSpec((hid, k), lambda i: (0, 0)),
      ],
      out_specs=[
          pl.BlockSpec((br, hid // 2), lambda i: (i, 0)),
          pl.BlockSpec((br, hid // 2), lambda i: (i, 0)),
          pl.BlockSpec((br, 16), lambda i: (i, 0)),
      ],
      out_shape=[
          jax.ShapeDtypeStruct((npad, hid // 2), jnp.float32),
          jax.ShapeDtypeStruct((npad, hid // 2), jnp.float32),
          jax.ShapeDtypeStruct((npad, 16), jnp.float32),
      ],
  )(x_p, d0, d1, w1)


def _prep2(s1, g1lo, g1hi, dinv, b1, w2, npad, br, hid, lat):
  hw = hid // 2

  def body(s00_ref, s01_ref, s10_ref, s11_ref, glo_ref, ghi_ref, dinv_ref,
           b_ref, w_ref, out_ref):
    dv = dinv_ref[...][:, 0:1]
    b = b_ref[...]
    zlo = dv * (s00_ref[...] + s10_ref[...] + glo_ref[...]) + b[:, :hw]
    zhi = dv * (s01_ref[...] + s11_ref[...] + ghi_ref[...]) + b[:, hw:]
    zlo = jnp.maximum(zlo, 0.0) * dv
    zhi = jnp.maximum(zhi, 0.0) * dv
    w = w_ref[...]
    out_ref[...] = (
        lax.dot_general(zlo, w[:, :hw], (((1,), (1,)), ((), ())),
                        preferred_element_type=jnp.float32) +
        lax.dot_general(zhi, w[:, hw:], (((1,), (1,)), ((), ())),
                        preferred_element_type=jnp.float32))

  return pl.pallas_call(
      body,
      grid=(npad // br,),
      in_specs=[
          pl.BlockSpec((br, hw), lambda i: (i, 0)),
          pl.BlockSpec((br, hw), lambda i: (i, 0)),
          pl.BlockSpec((br, hw), lambda i: (i, 0)),
          pl.BlockSpec((br, hw), lambda i: (i, 0)),
          pl.BlockSpec((br, hw), lambda i: (i, 0)),
          pl.BlockSpec((br, hw), lambda i: (i, 0)),
          pl.BlockSpec((br, 16), lambda i: (i, 0)),
          pl.BlockSpec((1, hid), lambda i: (0, 0)),
          pl.BlockSpec((lat, hid), lambda i: (0, 0)),
      ],
      out_specs=pl.BlockSpec((br, lat), lambda i: (i, 0)),
      out_shape=jax.ShapeDtypeStruct((npad, lat), jnp.float32),
  )(s1[0, 0], s1[0, 1], s1[1, 0], s1[1, 1], g1lo, g1hi, dinv, b1, w2)


def _finalize(s0, s1, g2, dinv, b2, npad, br, lat):
  def body(s0_ref, s1_ref, g2_ref, dinv_ref, b_ref, out_ref):
    dv = dinv_ref[...][:, 0:1]
    out_ref[...] = dv * (s0_ref[...] + s1_ref[...] + g2_ref[...]) + b_ref[...]

  return pl.pallas_call(
      body,
      grid=(npad // br,),
      in_specs=[
          pl.BlockSpec((br, lat), lambda i: (i, 0)),
          pl.BlockSpec((br, lat), lambda i: (i, 0)),
          pl.BlockSpec((br, lat), lambda i: (i, 0)),
          pl.BlockSpec((br, 16), lambda i: (i, 0)),
          pl.BlockSpec((1, lat), lambda i: (0, 0)),
      ],
      out_specs=pl.BlockSpec((br, lat), lambda i: (i, 0)),
      out_shape=jax.ShapeDtypeStruct((npad, lat), jnp.float32),
  )(s0, s1, g2, dinv, b2)


def kernel(x, edge_index, W1, b1, W2, b2):
  n, k = x.shape
  hid = W1.shape[0]
  lat = W2.shape[0]

  src = edge_index[0].astype(jnp.int32)
  dst = edge_index[1].astype(jnp.int32)
  e = src.shape[0]

  ept = -(-e // (NW * 384)) * 384
  epad = ept * NW
  nchunks = ept // CH
  src_p = jnp.concatenate([src, jnp.full((epad - e,), n, jnp.int32)])
  dst_p = jnp.concatenate([dst, jnp.full((epad - e,), n, jnp.int32)])
  def chunked(a, ch, extra):
    r = a.reshape(NW, ept // ch, ch)
    if extra:
      r = jnp.concatenate([r, jnp.full((NW, extra, ch), n, jnp.int32)], axis=1)
    return r

  rpt = -(-(n + 1) // NS)
  rpt = -(-rpt // 8) * 8
  npad = rpt * NS
  br = rpt

  x_p = jnp.pad(x, ((0, npad - n), (0, 0)))

  ach = 64
  src_c = chunked(src_p, ach, 0)
  dst_c = chunked(dst_p, ach, 0)
  deg = _make_deg_kernel(npad, rpt, nchunks)(chunked(dst_p, CH, 0))
  g1lo, g1hi, dinv = _prep1(x_p, deg[0], deg[1], W1, npad, br, k, hid)
  s1 = _make_agg_kernel(npad, rpt, ept // ach, 2, ach)(g1lo, g1hi, src_c,
                                                       dst_c)
  g2 = _prep2(s1, g1lo, g1hi, dinv, b1.reshape(1, hid), W2, npad, br, hid,
              lat)
  s2 = _make_agg_kernel(npad, rpt, ept // ach, 1, ach)(g2, src_c, dst_c)
  out = _finalize(s2[0, 0], s2[1, 0], g2, dinv, b2.reshape(1, lat), npad, br,
                  lat)
  return out[:n]


# speedup vs baseline: 1.0121x; 1.0121x over previous
import functools

import jax
import jax.numpy as jnp
from jax import lax
from jax.experimental import pallas as pl
from jax.experimental.pallas import tpu as pltpu
from jax.experimental.pallas import tpu_sc as plsc

NC = 2
NS = 16
NW = NC * NS
CH = 128


def _zero_rows(zbuf, nrows, width):
  z16 = jnp.zeros((16,), jnp.float32)

  def body(i, c):
    for j in range(width // 16):
      zbuf[i, pl.ds(j * 16, 16)] = z16
    return c

  lax.fori_loop(0, nrows, body, 0)


def _make_deg_kernel(npad, rpt, nchunks):
  mesh = plsc.VectorSubcoreMesh(core_axis_name="c", subcore_axis_name="s")
  zr = rpt // 4

  def body(dst_hbm, out_hbm, didx_all, ones_v, zbuf, deg_sh):
    cid = lax.axis_index("c")
    sid = lax.axis_index("s")
    wid = cid * NS + sid

    pltpu.sync_copy(dst_hbm.at[wid], didx_all)

    o16 = jnp.ones((16,), jnp.float32)

    def fill(i, c):
      ones_v[i, :] = o16
      return c

    lax.fori_loop(0, CH, fill, 0)
    _zero_rows(zbuf, zr, 16)

    def zcopy(i, c):
      pltpu.sync_copy(zbuf, deg_sh.at[pl.ds(sid * rpt + i * zr, zr)])
      return c

    lax.fori_loop(0, rpt // zr, zcopy, 0)
    plsc.subcore_barrier()

    def step(i, c):
      pltpu.sync_copy(ones_v, deg_sh.at[didx_all.at[i]], add=True)
      return c

    lax.fori_loop(0, nchunks, step, 0)
    plsc.subcore_barrier()
    pltpu.sync_copy(deg_sh.at[pl.ds(sid * rpt, rpt)],
                    out_hbm.at[cid, pl.ds(sid * rpt, rpt), :])

  return pl.kernel(
      body,
      out_type=jax.ShapeDtypeStruct((NC, npad, 16), jnp.float32),
      mesh=mesh,
      compiler_params=pltpu.CompilerParams(use_tc_tiling_on_sc=False),
      scratch_types=[
          pltpu.VMEM((nchunks, CH), jnp.int32),
          pltpu.VMEM((CH, 16), jnp.float32),
          pltpu.VMEM((zr, 16), jnp.float32),
          pltpu.VMEM_SHARED((npad, 16), jnp.float32),
      ],
  )


def _make_agg_kernel(npad, rpt, nchunks, nh, ch):
  mesh = plsc.VectorSubcoreMesh(core_axis_name="c", subcore_axis_name="s")
  hw = 64

  nb = 4
  gd = nb // 2
  assert nchunks % nb == 0 and nchunks > nb

  def body(*refs):
    g_halves = refs[:nh]
    src_hbm, dst_hbm, out_hbm = refs[nh:nh + 3]
    sidx_all, didx_all = refs[nh + 3:nh + 5]
    rows = refs[nh + 5:nh + 5 + nb]
    table, accum = refs[nh + 5 + nb:nh + 7 + nb]
    gsem = refs[nh + 7 + nb:nh + 7 + 2 * nb]
    ssem = refs[nh + 7 + 2 * nb:nh + 7 + 3 * nb]
    rows0 = rows[0]
    cid = lax.axis_index("c")
    sid = lax.axis_index("s")
    wid = cid * NS + sid

    pltpu.sync_copy(src_hbm.at[wid], sidx_all)
    pltpu.sync_copy(dst_hbm.at[wid], didx_all)

    for h in range(nh):
      if h:
        plsc.subcore_barrier()
      _zero_rows(rows0, ch, hw)
      pltpu.sync_copy(g_halves[h].at[pl.ds(sid * rpt, rpt)],
                      table.at[pl.ds(sid * rpt, rpt)])

      def zcopy(i, c):
        pltpu.sync_copy(rows0, accum.at[pl.ds(sid * rpt + i * ch, ch)])
        return c

      lax.fori_loop(0, rpt // ch, zcopy, 0)
      rem = rpt % ch
      if rem:
        pltpu.sync_copy(rows0.at[pl.ds(0, rem)],
                        accum.at[pl.ds(sid * rpt + (rpt // ch) * ch, rem)])
      plsc.subcore_barrier()

      def fire_gather(i, b):
        pltpu.async_copy(table.at[sidx_all.at[i]], rows[b], gsem[b])

      def wait_gather(i, b):
        pltpu.make_async_copy(table.at[sidx_all.at[i]], rows[b],
                              gsem[b]).wait()

      def fire_scatter(i, b):
        pltpu.async_copy(rows[b], accum.at[didx_all.at[i]], ssem[b],
                         add=True)

      def wait_scatter(i, b):
        pltpu.make_async_copy(rows[b], accum.at[didx_all.at[i]],
                              ssem[b]).wait()

      for i in range(gd):
        fire_gather(i, i)
      for i in range(nb - gd):
        wait_gather(i, i)
        fire_scatter(i, i)
        fire_gather(i + gd, (i + gd) % nb)

      def group(q, c):
        i0 = (nb - gd) + nb * q
        for j in range(nb):
          i = i0 + j
          b = (nb - gd + j) % nb
          wait_gather(i, b)
          fire_scatter(i, b)
          b2 = (b + gd) % nb
          wait_scatter(i + gd - nb, b2)
          fire_gather(i + gd, b2)
        return c

      lax.fori_loop(0, (nchunks - nb) // nb, group, 0)
      for i in range(nchunks - gd, nchunks):
        wait_gather(i, i % nb)
        fire_scatter(i, i % nb)
      for i in range(nchunks - nb, nchunks):
        wait_scatter(i, i % nb)
      plsc.subcore_barrier()
      pltpu.sync_copy(accum.at[pl.ds(sid * rpt, rpt)],
                      out_hbm.at[cid, h, pl.ds(sid * rpt, rpt), :])

  return pl.kernel(
      body,
      out_type=jax.ShapeDtypeStruct((NC, nh, npad, hw), jnp.float32),
      mesh=mesh,
      compiler_params=pltpu.CompilerParams(use_tc_tiling_on_sc=False),
      scratch_types=(
          [pltpu.VMEM((nchunks, ch), jnp.int32)] * 2 +
          [pltpu.VMEM((ch, hw), jnp.float32)] * nb +
          [pltpu.VMEM_SHARED((npad, hw), jnp.float32)] * 2 +
          [pltpu.SemaphoreType.DMA] * (2 * nb)),
  )


def _prep1(x_p, d0, d1, w1, npad, br, k, hid):
  def body(x_ref, d0_ref, d1_ref, w_ref, glo_ref, ghi_ref, dinv_ref):
    deg = d0_ref[...] + d1_ref[...] + 1.0
    dinv = lax.rsqrt(deg)
    dinv_ref[...] = dinv
    xs = x_ref[...] * dinv[:, 0:1]
    g = lax.dot_general(xs, w_ref[...], (((1,), (1,)), ((), ())),
                        preferred_element_type=jnp.float32)
    glo_ref[...] = g[:, :hid // 2]
    ghi_ref[...] = g[:, hid // 2:]

  return pl.pallas_call(
      body,
      grid=(npad // br,),
      in_specs=[
          pl.BlockSpec((br, k), lambda i: (i, 0)),
          pl.BlockSpec((br, 16), lambda i: (i, 0)),
          pl.BlockSpec((br, 16), lambda i: (i, 0)),
          pl.BlockSpec((hid, k), lambda i: (0, 0)),
      ],
      out_specs=[
          pl.BlockSpec((br, hid // 2), lambda i: (i, 0)),
          pl.BlockSpec((br, hid // 2), lambda i: (i, 0)),
          pl.BlockSpec((br, 16), lambda i: (i, 0)),
      ],
      out_shape=[
          jax.ShapeDtypeStruct((npad, hid // 2), jnp.float32),
          jax.ShapeDtypeStruct((npad, hid // 2), jnp.float32),
          jax.ShapeDtypeStruct((npad, 16), jnp.float32),
      ],
  )(x_p, d0, d1, w1)


def _prep2(s1, g1lo, g1hi, dinv, b1, w2, npad, br, hid, lat):
  hw = hid // 2

  def body(s00_ref, s01_ref, s10_ref, s11_ref, glo_ref, ghi_ref, dinv_ref,
           b_ref, w_ref, out_ref):
    dv = dinv_ref[...][:, 0:1]
    b = b_ref[...]
    zlo = dv * (s00_ref[...] + s10_ref[...] + glo_ref[...]) + b[:, :hw]
    zhi = dv * (s01_ref[...] + s11_ref[...] + ghi_ref[...]) + b[:, hw:]
    zlo = jnp.maximum(zlo, 0.0) * dv
    zhi = jnp.maximum(zhi, 0.0) * dv
    w = w_ref[...]
    out_ref[...] = (
        lax.dot_general(zlo, w[:, :hw], (((1,), (1,)), ((), ())),
                        preferred_element_type=jnp.float32) +
        lax.dot_general(zhi, w[:, hw:], (((1,), (1,)), ((), ())),
                        preferred_element_type=jnp.float32))

  return pl.pallas_call(
      body,
      grid=(npad // br,),
      in_specs=[
          pl.BlockSpec((br, hw), lambda i: (i, 0)),
          pl.BlockSpec((br, hw), lambda i: (i, 0)),
          pl.BlockSpec((br, hw), lambda i: (i, 0)),
          pl.BlockSpec((br, hw), lambda i: (i, 0)),
          pl.BlockSpec((br, hw), lambda i: (i, 0)),
          pl.BlockSpec((br, hw), lambda i: (i, 0)),
          pl.BlockSpec((br, 16), lambda i: (i, 0)),
          pl.BlockSpec((1, hid), lambda i: (0, 0)),
          pl.BlockSpec((lat, hid), lambda i: (0, 0)),
      ],
      out_specs=pl.BlockSpec((br, lat), lambda i: (i, 0)),
      out_shape=jax.ShapeDtypeStruct((npad, lat), jnp.float32),
  )(s1[0, 0], s1[0, 1], s1[1, 0], s1[1, 1], g1lo, g1hi, dinv, b1, w2)


def _finalize(s0, s1, g2, dinv, b2, npad, br, lat):
  def body(s0_ref, s1_ref, g2_ref, dinv_ref, b_ref, out_ref):
    dv = dinv_ref[...][:, 0:1]
    out_ref[...] = dv * (s0_ref[...] + s1_ref[...] + g2_ref[...]) + b_ref[...]

  return pl.pallas_call(
      body,
      grid=(npad // br,),
      in_specs=[
          pl.BlockSpec((br, lat), lambda i: (i, 0)),
          pl.BlockSpec((br, lat), lambda i: (i, 0)),
          pl.BlockSpec((br, lat), lambda i: (i, 0)),
          pl.BlockSpec((br, 16), lambda i: (i, 0)),
          pl.BlockSpec((1, lat), lambda i: (0, 0)),
      ],
      out_specs=pl.BlockSpec((br, lat), lambda i: (i, 0)),
      out_shape=jax.ShapeDtypeStruct((npad, lat), jnp.float32),
  )(s0, s1, g2, dinv, b2)


def kernel(x, edge_index, W1, b1, W2, b2):
  n, k = x.shape
  hid = W1.shape[0]
  lat = W2.shape[0]

  src = edge_index[0].astype(jnp.int32)
  dst = edge_index[1].astype(jnp.int32)
  e = src.shape[0]

  ept = -(-e // (NW * 256)) * 256
  epad = ept * NW
  nchunks = ept // CH
  src_p = jnp.concatenate([src, jnp.full((epad - e,), n, jnp.int32)])
  dst_p = jnp.concatenate([dst, jnp.full((epad - e,), n, jnp.int32)])
  def chunked(a, ch, extra):
    r = a.reshape(NW, ept // ch, ch)
    if extra:
      r = jnp.concatenate([r, jnp.full((NW, extra, ch), n, jnp.int32)], axis=1)
    return r

  rpt = -(-(n + 1) // NS)
  rpt = -(-rpt // 8) * 8
  npad = rpt * NS
  br = rpt

  x_p = jnp.pad(x, ((0, npad - n), (0, 0)))

  ach = 64
  src_c = chunked(src_p, ach, 0)
  dst_c = chunked(dst_p, ach, 0)
  deg = _make_deg_kernel(npad, rpt, nchunks)(chunked(dst_p, CH, 0))
  g1lo, g1hi, dinv = _prep1(x_p, deg[0], deg[1], W1, npad, br, k, hid)
  s1 = _make_agg_kernel(npad, rpt, ept // ach, 2, ach)(g1lo, g1hi, src_c,
                                                       dst_c)
  g2 = _prep2(s1, g1lo, g1hi, dinv, b1.reshape(1, hid), W2, npad, br, hid,
              lat)
  s2 = _make_agg_kernel(npad, rpt, ept // ach, 1, ach)(g2, src_c, dst_c)
  out = _finalize(s2[0, 0], s2[1, 0], g2, dinv, b2.reshape(1, lat), npad, br,
                  lat)
  return out[:n]
